# baseline (speedup 1.0000x reference)
import functools

import jax
import jax.numpy as jnp
from jax import lax
from jax.experimental import pallas as pl
from jax.experimental.pallas import tpu as pltpu
from jax.experimental.pallas import tpu_sc as plsc

N = 5000
E = 80000
F_IN = 256
H = 128
O = 96
OP = 128

NC = 2
NS = 16
CLEN = 64
_DEPTH = 8
CH = -(-(-(-E // CLEN) // NS) // _DEPTH) * _DEPTH
CGP = NS * CH
EPAD = CGP * CLEN
NPAD = -(-N // (NS * 8)) * NS * 8
RPT = NPAD // NS

BR = 1000
NBLK = N // BR


def _seg_body(q_hbm, src_hbm, dst_hbm, zero_hbm, out_hbm, idx_s, idx_d,
              *rest):
    rows = list(rest[:_DEPTH])
    acc = rest[_DEPTH]
    gsem = list(rest[_DEPTH + 1:2 * _DEPTH + 1])
    ssem = list(rest[2 * _DEPTH + 1:])
    rows0 = rows[0]
    c = lax.axis_index("c")
    s = lax.axis_index("s")

    pltpu.sync_copy(src_hbm.at[c, s], idx_s)
    pltpu.sync_copy(dst_hbm.at[c, s], idx_d)
    pltpu.sync_copy(zero_hbm, rows0)
    full, rem = divmod(RPT, CLEN)
    for b in range(full):
        pltpu.sync_copy(rows0, acc.at[pl.ds(s * RPT + b * CLEN, CLEN)])
    if rem:
        pltpu.sync_copy(rows0.at[pl.ds(0, rem)],
                        acc.at[pl.ds(s * RPT + full * CLEN, rem)])
    plsc.subcore_barrier()

    def _gather(j, b):
        return pltpu.async_copy(q_hbm.at[idx_s.at[j]], rows[b], gsem[b])

    def _gather_wait(j, b):
        pltpu.make_async_copy(q_hbm.at[idx_s.at[j]], rows[b], gsem[b]).wait()

    def _scatter(j, b):
        return pltpu.async_copy(rows[b], acc.at[idx_d.at[j]], ssem[b],
                                add=True)

    def _scatter_wait(j, b):
        pltpu.make_async_copy(rows[b], acc.at[idx_d.at[j]], ssem[b]).wait()

    for k in range(_DEPTH - 1):
        _gather(k, k)
    _gather_wait(0, 0)
    _scatter(0, 0)
    _gather(_DEPTH - 1, _DEPTH - 1)

    def _step(t, carry):
        j0 = 1 + t * _DEPTH
        for u in range(_DEPTH):
            j = j0 + u
            b = (1 + u) % _DEPTH
            bn = u % _DEPTH
            _gather_wait(j, b)
            _scatter(j, b)
            _scatter_wait(j - 1, bn)
            _gather(j + _DEPTH - 1, bn)
        return carry

    nsteps = (CH - _DEPTH) // _DEPTH
    lax.fori_loop(0, nsteps, _step, 0)
    for j in range(CH - _DEPTH + 1, CH):
        _gather_wait(j, j % _DEPTH)
        _scatter(j, j % _DEPTH)
    for j in range(CH - _DEPTH, CH):
        _scatter_wait(j, j % _DEPTH)

    plsc.subcore_barrier()
    pltpu.sync_copy(acc.at[pl.ds(s * RPT, RPT)],
                    out_hbm.at[c, pl.ds(s * RPT, RPT)])


@functools.cache
def _make_segment_sum_sc(F):
    mesh = plsc.VectorSubcoreMesh(core_axis_name="c", subcore_axis_name="s")
    return functools.partial(
        pl.kernel,
        out_type=jax.ShapeDtypeStruct((NC, NPAD, F), jnp.float32),
        mesh=mesh,
        scratch_types=(
            [pltpu.VMEM((CH, CLEN), jnp.int32)] * 2
            + [pltpu.VMEM((CLEN, F), jnp.float32)] * _DEPTH
            + [pltpu.VMEM_SHARED((NPAD, F), jnp.float32)]
            + [pltpu.SemaphoreType.DMA] * (2 * _DEPTH)
        ),
    )(_seg_body)


def _segment_sum_sc(q, src4d, dst4d, zeros, F):
    return _make_segment_sum_sc(F)(q, src4d, dst4d, zeros)


def _mm0_body(x_ref, ws_ref, wn_ref, b_ref, p_ref, q_ref):
    x = x_ref[...]
    p_ref[...] = (jnp.dot(x, ws_ref[...], preferred_element_type=jnp.float32)
                  + b_ref[...])
    q_ref[...] = jnp.dot(x, wn_ref[...], preferred_element_type=jnp.float32)


def _mid_body(p_ref, s_ref, ws_ref, wn_ref, b_ref, po_ref, qo_ref):
    h = jnp.maximum(p_ref[...] + s_ref[0], 0.0)
    po_ref[...] = (jnp.dot(h, ws_ref[...], preferred_element_type=jnp.float32)
                   + b_ref[...])
    qo_ref[...] = jnp.dot(h, wn_ref[...], preferred_element_type=jnp.float32)


def _fin_body(p_ref, s_ref, h_ref, g_ref):
    h = jnp.maximum(p_ref[...] + s_ref[0], 0.0)
    h_ref[...] = h
    col = jnp.sum(h, axis=0, keepdims=True)
    i = pl.program_id(0)

    @pl.when(i == 0)
    def _():
        g_ref[...] = jnp.zeros((2, OP), jnp.float32)

    row = jax.lax.broadcasted_iota(jnp.int32, (2, OP), 0)
    g_ref[...] = g_ref[...] + jnp.where(row == i // NBLK, col, 0.0)


def _attn_body(hs_ref, ht_ref, o_ref):
    sc = jax.lax.dot_general(hs_ref[...], ht_ref[...],
                             (((1,), (1,)), ((), ())),
                             preferred_element_type=jnp.float32)
    sc = sc * (1.0 / (O ** 0.5))
    m = jnp.max(sc, axis=-1, keepdims=True)
    e = jnp.exp(sc - m)
    o_ref[...] = e / jnp.sum(e, axis=-1, keepdims=True)


def _head_body(g_ref, wa_ref, wb_ref, b1_ref, w2_ref, b2_ref, sim_ref, lg_ref):
    g = g_ref[...] * (1.0 / N)
    gs = g[0:1, :]
    gt = g[1:2, :]
    z = jnp.maximum(
        jnp.dot(gs, wa_ref[...], preferred_element_type=jnp.float32)
        + jnp.dot(gt, wb_ref[...], preferred_element_type=jnp.float32)
        + b1_ref[...], 0.0)
    lg_ref[...] = (jnp.dot(z, w2_ref[...], preferred_element_type=jnp.float32)
                   + b2_ref[...])
    num = jnp.sum(gs * gt, axis=1, keepdims=True)
    ns = jnp.sqrt(jnp.sum(gs * gs, axis=1, keepdims=True))
    nt = jnp.sqrt(jnp.sum(gt * gt, axis=1, keepdims=True))
    sim_ref[...] = num / (ns * nt + 1e-8)


def _full(shape):
    return pl.BlockSpec(shape, lambda i: tuple(0 for _ in shape))


def _mm0(x2, Ws, Wn, b):
    return pl.pallas_call(
        _mm0_body,
        grid=(2 * NBLK,),
        in_specs=[pl.BlockSpec((BR, F_IN), lambda i: (i, 0)),
                  _full((F_IN, H)), _full((F_IN, H)), _full((1, H))],
        out_specs=[pl.BlockSpec((BR, H), lambda i: (i, 0)),
                   pl.BlockSpec((BR, H), lambda i: (i, 0))],
        out_shape=[jax.ShapeDtypeStruct((2 * N, H), jnp.float32)] * 2,
    )(x2, Ws, Wn, b.reshape(1, H))


def _mm_mid(p, seg, Ws, Wn, b, F_in, F_out):
    return pl.pallas_call(
        _mid_body,
        grid=(2 * NBLK,),
        in_specs=[pl.BlockSpec((BR, F_in), lambda i: (i, 0)),
                  pl.BlockSpec((1, BR, F_in),
                               lambda i: (i // NBLK, i % NBLK, 0)),
                  _full((F_in, F_out)), _full((F_in, F_out)),
                  _full((1, F_out))],
        out_specs=[pl.BlockSpec((BR, F_out), lambda i: (i, 0)),
                   pl.BlockSpec((BR, F_out), lambda i: (i, 0))],
        out_shape=[jax.ShapeDtypeStruct((2 * N, F_out), jnp.float32)] * 2,
    )(p, seg, Ws, Wn, b.reshape(1, F_out))


def _finalize(p, seg):
    return pl.pallas_call(
        _fin_body,
        grid=(2 * NBLK,),
        in_specs=[pl.BlockSpec((BR, OP), lambda i: (i, 0)),
                  pl.BlockSpec((1, BR, OP),
                               lambda i: (i // NBLK, i % NBLK, 0))],
        out_specs=[pl.BlockSpec((BR, OP), lambda i: (i, 0)),
                   pl.BlockSpec((2, OP), lambda i: (0, 0))],
        out_shape=[jax.ShapeDtypeStruct((2 * N, OP), jnp.float32),
                   jax.ShapeDtypeStruct((2, OP), jnp.float32)],
    )(p, seg)


BR_AT = 200


def _attention(h):
    return pl.pallas_call(
        _attn_body,
        grid=(N // BR_AT,),
        in_specs=[pl.BlockSpec((BR_AT, OP), lambda i: (i, 0)),
                  pl.BlockSpec((N, OP), lambda i: (1, 0))],
        out_specs=pl.BlockSpec((BR_AT, N), lambda i: (i, 0)),
        out_shape=jax.ShapeDtypeStruct((N, N), jnp.float32),
    )(h, h)


def _head(gsum, Wc1, bc1, Wc2, bc2):
    pad = ((0, OP - O), (0, 0))
    return pl.pallas_call(
        _head_body,
        out_shape=[jax.ShapeDtypeStruct((1, 1), jnp.float32),
                   jax.ShapeDtypeStruct((1, 4), jnp.float32)],
    )(gsum, jnp.pad(Wc1[:O], pad), jnp.pad(Wc1[O:], pad),
      bc1.reshape(1, H), Wc2, bc2.reshape(1, 4))


def kernel(x_s, edge_index_s, x_t, edge_index_t,
           W0s, W0n, b0, W1s, W1n, b1, W2s, W2n, b2,
           Wc1, bc1, Wc2, bc2):
    x2 = jnp.concatenate([x_s, x_t], axis=0)

    def _prep(edge_index, node_off):
        src = edge_index[0].astype(jnp.int32) + node_off
        dst = edge_index[1].astype(jnp.int32)
        perm = jnp.argsort(src)
        src = src[perm]
        dst = dst[perm]
        src_p = jnp.concatenate(
            [src, jnp.full((EPAD - E,), node_off, jnp.int32)])
        dst_p = jnp.concatenate([dst, jnp.full((EPAD - E,), N, jnp.int32)])
        return src_p.reshape(NS, CH, CLEN), dst_p.reshape(NS, CH, CLEN)

    ss, ds = _prep(edge_index_s, 0)
    st, dt = _prep(edge_index_t, N)
    src4d = jnp.stack([ss, st])
    dst4d = jnp.stack([ds, dt])

    zeros = jnp.zeros((CLEN, H), jnp.float32)
    wpad = ((0, 0), (0, OP - O))
    p0, q0 = _mm0(x2, W0s, W0n, b0)
    seg0 = _segment_sum_sc(q0, src4d, dst4d, zeros, H)
    p1, q1 = _mm_mid(p0, seg0, W1s, W1n, b1, H, H)
    seg1 = _segment_sum_sc(q1, src4d, dst4d, zeros, H)
    p2, q2 = _mm_mid(p1, seg1, jnp.pad(W2s, wpad), jnp.pad(W2n, wpad),
                     jnp.pad(b2, (0, OP - O)), H, OP)
    seg2 = _segment_sum_sc(q2, src4d, dst4d, zeros, OP)
    h, gsum = _finalize(p2, seg2)

    attn = _attention(h)
    sim, logits = _head(gsum, Wc1, bc1, Wc2, bc2)
    return sim.reshape(()), attn, logits.reshape(4)


# speedup vs baseline: 2.7570x; 2.7570x over previous
import functools

import jax
import jax.numpy as jnp
from jax import lax
from jax.experimental import pallas as pl
from jax.experimental.pallas import tpu as pltpu
from jax.experimental.pallas import tpu_sc as plsc

N = 5000
E = 80000
F_IN = 256
H = 128
O = 96
OP = 128

NC = 2
NS = 16
CLEN = 128
_DEPTH = 2
CH = -(-(-(-E // CLEN) // NS) // _DEPTH) * _DEPTH
CGP = NS * CH
EPAD = CGP * CLEN
NPAD = -(-N // (NS * 8)) * NS * 8
RPT = NPAD // NS

BR = 1000
NBLK = N // BR


def _seg_body(q_hbm, src_hbm, dst_hbm, zero_hbm, out_hbm, idx_s, idx_d,
              *rest):
    rows = list(rest[:_DEPTH])
    acc = rest[_DEPTH]
    q_sp = rest[_DEPTH + 1]
    zsem = rest[_DEPTH + 2]
    gsem = list(rest[_DEPTH + 3:2 * _DEPTH + 3])
    ssem = list(rest[2 * _DEPTH + 3:])
    rows0 = rows[0]
    c = lax.axis_index("c")
    s = lax.axis_index("s")

    pltpu.sync_copy(src_hbm.at[c, s], idx_s)
    pltpu.sync_copy(dst_hbm.at[c, s], idx_d)

    pltpu.sync_copy(zero_hbm, rows0)
    full, rem = divmod(RPT, CLEN)
    for b in range(full):
        pltpu.async_copy(rows0, acc.at[pl.ds(s * RPT + b * CLEN, CLEN)], zsem)
    if rem:
        pltpu.async_copy(rows0.at[pl.ds(0, rem)],
                         acc.at[pl.ds(s * RPT + full * CLEN, rem)], zsem)
    REM = N - (NS - 1) * RPT

    @pl.when(s < NS - 1)
    def _():
        pltpu.sync_copy(q_hbm.at[pl.ds(c * N + s * RPT, RPT)],
                        q_sp.at[pl.ds(s * RPT, RPT)])

    @pl.when(s == NS - 1)
    def _():
        pltpu.sync_copy(q_hbm.at[pl.ds(c * N + (NS - 1) * RPT, REM)],
                        q_sp.at[pl.ds((NS - 1) * RPT, REM)])

    for b in range(full):
        pltpu.make_async_copy(
            rows0, acc.at[pl.ds(s * RPT + b * CLEN, CLEN)], zsem).wait()
    if rem:
        pltpu.make_async_copy(
            rows0.at[pl.ds(0, rem)],
            acc.at[pl.ds(s * RPT + full * CLEN, rem)], zsem).wait()
    plsc.subcore_barrier()

    def _gather(j, b):
        return pltpu.async_copy(q_sp.at[idx_s.at[j]], rows[b], gsem[b])

    def _gather_wait(j, b):
        pltpu.make_async_copy(q_sp.at[idx_s.at[j]], rows[b], gsem[b]).wait()

    def _scatter(j, b):
        return pltpu.async_copy(rows[b], acc.at[idx_d.at[j]], ssem[b],
                                add=True)

    def _scatter_wait(j, b):
        pltpu.make_async_copy(rows[b], acc.at[idx_d.at[j]], ssem[b]).wait()

    for k in range(_DEPTH - 1):
        _gather(k, k)
    _gather_wait(0, 0)
    _scatter(0, 0)
    _gather(_DEPTH - 1, _DEPTH - 1)

    def _step(t, carry):
        j0 = 1 + t * _DEPTH
        for u in range(_DEPTH):
            j = j0 + u
            b = (1 + u) % _DEPTH
            bn = u % _DEPTH
            _gather_wait(j, b)
            _scatter(j, b)
            _scatter_wait(j - 1, bn)
            _gather(j + _DEPTH - 1, bn)
        return carry

    nsteps = (CH - _DEPTH) // _DEPTH
    lax.fori_loop(0, nsteps, _step, 0)
    for j in range(CH - _DEPTH + 1, CH):
        _gather_wait(j, j % _DEPTH)
        _scatter(j, j % _DEPTH)
    for j in range(CH - _DEPTH, CH):
        _scatter_wait(j, j % _DEPTH)

    plsc.subcore_barrier()
    pltpu.sync_copy(acc.at[pl.ds(s * RPT, RPT)],
                    out_hbm.at[c, pl.ds(s * RPT, RPT)])


@functools.cache
def _make_segment_sum_sc(F):
    mesh = plsc.VectorSubcoreMesh(core_axis_name="c", subcore_axis_name="s")
    return functools.partial(
        pl.kernel,
        out_type=jax.ShapeDtypeStruct((NC, NPAD, F), jnp.float32),
        mesh=mesh,
        scratch_types=(
            [pltpu.VMEM((CH, CLEN), jnp.int32)] * 2
            + [pltpu.VMEM((CLEN, F), jnp.float32)] * _DEPTH
            + [pltpu.VMEM_SHARED((NPAD, F), jnp.float32),
               pltpu.VMEM_SHARED((N, F), jnp.float32)]
            + [pltpu.SemaphoreType.DMA] * (2 * _DEPTH + 1)
        ),
    )(_seg_body)


def _segment_sum_sc(q, src4d, dst4d, zeros, F):
    return _make_segment_sum_sc(F)(q, src4d, dst4d, zeros)


def _mm0_body(x_ref, ws_ref, wn_ref, b_ref, p_ref, q_ref):
    x = x_ref[...]
    p_ref[...] = (jnp.dot(x, ws_ref[...], preferred_element_type=jnp.float32)
                  + b_ref[...])
    q_ref[...] = jnp.dot(x, wn_ref[...], preferred_element_type=jnp.float32)


def _mid_body(p_ref, s_ref, ws_ref, wn_ref, b_ref, po_ref, qo_ref):
    h = jnp.maximum(p_ref[...] + s_ref[0], 0.0)
    po_ref[...] = (jnp.dot(h, ws_ref[...], preferred_element_type=jnp.float32)
                   + b_ref[...])
    qo_ref[...] = jnp.dot(h, wn_ref[...], preferred_element_type=jnp.float32)


def _fin_body(p_ref, s_ref, h_ref, g_ref):
    h = jnp.maximum(p_ref[...] + s_ref[0], 0.0)
    h_ref[...] = h
    col = jnp.sum(h, axis=0, keepdims=True)
    i = pl.program_id(0)

    @pl.when(i == 0)
    def _():
        g_ref[...] = jnp.zeros((2, OP), jnp.float32)

    row = jax.lax.broadcasted_iota(jnp.int32, (2, OP), 0)
    g_ref[...] = g_ref[...] + jnp.where(row == i // NBLK, col, 0.0)


def _attn_body(hs_ref, ht_ref, o_ref):
    sc = jax.lax.dot_general(hs_ref[...], ht_ref[...],
                             (((1,), (1,)), ((), ())),
                             preferred_element_type=jnp.float32)
    sc = sc * (1.0 / (O ** 0.5))
    m = jnp.max(sc, axis=-1, keepdims=True)
    e = jnp.exp(sc - m)
    o_ref[...] = e / jnp.sum(e, axis=-1, keepdims=True)


def _head_body(g_ref, wa_ref, wb_ref, b1_ref, w2_ref, b2_ref, sim_ref, lg_ref):
    g = g_ref[...] * (1.0 / N)
    gs = g[0:1, :]
    gt = g[1:2, :]
    z = jnp.maximum(
        jnp.dot(gs, wa_ref[...], preferred_element_type=jnp.float32)
        + jnp.dot(gt, wb_ref[...], preferred_element_type=jnp.float32)
        + b1_ref[...], 0.0)
    lg_ref[...] = (jnp.dot(z, w2_ref[...], preferred_element_type=jnp.float32)
                   + b2_ref[...])
    num = jnp.sum(gs * gt, axis=1, keepdims=True)
    ns = jnp.sqrt(jnp.sum(gs * gs, axis=1, keepdims=True))
    nt = jnp.sqrt(jnp.sum(gt * gt, axis=1, keepdims=True))
    sim_ref[...] = num / (ns * nt + 1e-8)


def _full(shape):
    return pl.BlockSpec(shape, lambda i: tuple(0 for _ in shape))


def _mm0(x2, Ws, Wn, b):
    return pl.pallas_call(
        _mm0_body,
        grid=(2 * NBLK,),
        in_specs=[pl.BlockSpec((BR, F_IN), lambda i: (i, 0)),
                  _full((F_IN, H)), _full((F_IN, H)), _full((1, H))],
        out_specs=[pl.BlockSpec((BR, H), lambda i: (i, 0)),
                   pl.BlockSpec((BR, H), lambda i: (i, 0))],
        out_shape=[jax.ShapeDtypeStruct((2 * N, H), jnp.float32)] * 2,
    )(x2, Ws, Wn, b.reshape(1, H))


def _mm_mid(p, seg, Ws, Wn, b, F_in, F_out):
    return pl.pallas_call(
        _mid_body,
        grid=(2 * NBLK,),
        in_specs=[pl.BlockSpec((BR, F_in), lambda i: (i, 0)),
                  pl.BlockSpec((1, BR, F_in),
                               lambda i: (i // NBLK, i % NBLK, 0)),
                  _full((F_in, F_out)), _full((F_in, F_out)),
                  _full((1, F_out))],
        out_specs=[pl.BlockSpec((BR, F_out), lambda i: (i, 0)),
                   pl.BlockSpec((BR, F_out), lambda i: (i, 0))],
        out_shape=[jax.ShapeDtypeStruct((2 * N, F_out), jnp.float32)] * 2,
    )(p, seg, Ws, Wn, b.reshape(1, F_out))


def _finalize(p, seg):
    return pl.pallas_call(
        _fin_body,
        grid=(2 * NBLK,),
        in_specs=[pl.BlockSpec((BR, OP), lambda i: (i, 0)),
                  pl.BlockSpec((1, BR, OP),
                               lambda i: (i // NBLK, i % NBLK, 0))],
        out_specs=[pl.BlockSpec((BR, OP), lambda i: (i, 0)),
                   pl.BlockSpec((2, OP), lambda i: (0, 0))],
        out_shape=[jax.ShapeDtypeStruct((2 * N, OP), jnp.float32),
                   jax.ShapeDtypeStruct((2, OP), jnp.float32)],
    )(p, seg)


BR_AT = 200


def _attention(h):
    return pl.pallas_call(
        _attn_body,
        grid=(N // BR_AT,),
        in_specs=[pl.BlockSpec((BR_AT, OP), lambda i: (i, 0)),
                  pl.BlockSpec((N, OP), lambda i: (1, 0))],
        out_specs=pl.BlockSpec((BR_AT, N), lambda i: (i, 0)),
        out_shape=jax.ShapeDtypeStruct((N, N), jnp.float32),
    )(h, h)


def _head(gsum, Wc1, bc1, Wc2, bc2):
    pad = ((0, OP - O), (0, 0))
    return pl.pallas_call(
        _head_body,
        out_shape=[jax.ShapeDtypeStruct((1, 1), jnp.float32),
                   jax.ShapeDtypeStruct((1, 4), jnp.float32)],
    )(gsum, jnp.pad(Wc1[:O], pad), jnp.pad(Wc1[O:], pad),
      bc1.reshape(1, H), Wc2, bc2.reshape(1, 4))


def kernel(x_s, edge_index_s, x_t, edge_index_t,
           W0s, W0n, b0, W1s, W1n, b1, W2s, W2n, b2,
           Wc1, bc1, Wc2, bc2):
    x2 = jnp.concatenate([x_s, x_t], axis=0)

    def _prep(edge_index):
        src = edge_index[0].astype(jnp.int32)
        dst = edge_index[1].astype(jnp.int32)
        src_p = jnp.concatenate([src, jnp.zeros((EPAD - E,), jnp.int32)])
        dst_p = jnp.concatenate([dst, jnp.full((EPAD - E,), N, jnp.int32)])
        return src_p.reshape(NS, CH, CLEN), dst_p.reshape(NS, CH, CLEN)

    ss, ds = _prep(edge_index_s)
    st, dt = _prep(edge_index_t)
    src4d = jnp.stack([ss, st])
    dst4d = jnp.stack([ds, dt])

    zeros = jnp.zeros((CLEN, H), jnp.float32)
    wpad = ((0, 0), (0, OP - O))
    p0, q0 = _mm0(x2, W0s, W0n, b0)
    seg0 = _segment_sum_sc(q0, src4d, dst4d, zeros, H)
    p1, q1 = _mm_mid(p0, seg0, W1s, W1n, b1, H, H)
    seg1 = _segment_sum_sc(q1, src4d, dst4d, zeros, H)
    p2, q2 = _mm_mid(p1, seg1, jnp.pad(W2s, wpad), jnp.pad(W2n, wpad),
                     jnp.pad(b2, (0, OP - O)), H, OP)
    seg2 = _segment_sum_sc(q2, src4d, dst4d, zeros, OP)
    h, gsum = _finalize(p2, seg2)

    attn = _attention(h)
    sim, logits = _head(gsum, Wc1, bc1, Wc2, bc2)
    return sim.reshape(()), attn, logits.reshape(4)


# speedup vs baseline: 2.9019x; 1.0526x over previous
import functools

import jax
import jax.numpy as jnp
from jax import lax
from jax.experimental import pallas as pl
from jax.experimental.pallas import tpu as pltpu
from jax.experimental.pallas import tpu_sc as plsc

N = 5000
E = 80000
F_IN = 256
H = 128
O = 96
OP = 128

NC = 2
NS = 16
CLEN = 64
_DEPTH = 3
_CG = -(-E // CLEN)
CH = -(-(-(-_CG // NS)) // _DEPTH) * _DEPTH
CGP = NS * CH
EPAD = CGP * CLEN
NPAD = -(-N // (NS * 8)) * NS * 8
RPT = NPAD // NS

BR = 1000
NBLK = N // BR


def _seg_body(q_hbm, src_hbm, dst_hbm, zero_hbm, out_hbm, idx_s, idx_d,
              *rest):
    rows = list(rest[:_DEPTH])
    acc = rest[_DEPTH]
    q_sp = rest[_DEPTH + 1]
    zsem = rest[_DEPTH + 2]
    gsem = list(rest[_DEPTH + 3:2 * _DEPTH + 3])
    ssem = list(rest[2 * _DEPTH + 3:])
    rows0 = rows[0]
    c = lax.axis_index("c")
    s = lax.axis_index("s")

    pltpu.sync_copy(src_hbm.at[c, s], idx_s)
    pltpu.sync_copy(dst_hbm.at[c, s], idx_d)

    pltpu.sync_copy(zero_hbm, rows0)
    full, rem = divmod(RPT, CLEN)
    for b in range(full):
        pltpu.async_copy(rows0, acc.at[pl.ds(s * RPT + b * CLEN, CLEN)], zsem)
    if rem:
        pltpu.async_copy(rows0.at[pl.ds(0, rem)],
                         acc.at[pl.ds(s * RPT + full * CLEN, rem)], zsem)
    REM = N - (NS - 1) * RPT

    @pl.when(s < NS - 1)
    def _():
        pltpu.sync_copy(q_hbm.at[pl.ds(c * N + s * RPT, RPT)],
                        q_sp.at[pl.ds(s * RPT, RPT)])

    @pl.when(s == NS - 1)
    def _():
        pltpu.sync_copy(q_hbm.at[pl.ds(c * N + (NS - 1) * RPT, REM)],
                        q_sp.at[pl.ds((NS - 1) * RPT, REM)])

    for b in range(full):
        pltpu.make_async_copy(
            rows0, acc.at[pl.ds(s * RPT + b * CLEN, CLEN)], zsem).wait()
    if rem:
        pltpu.make_async_copy(
            rows0.at[pl.ds(0, rem)],
            acc.at[pl.ds(s * RPT + full * CLEN, rem)], zsem).wait()
    plsc.subcore_barrier()

    def _gather(j, b):
        return pltpu.async_copy(q_sp.at[idx_s.at[j]], rows[b], gsem[b])

    def _gather_wait(j, b):
        pltpu.make_async_copy(q_sp.at[idx_s.at[j]], rows[b], gsem[b]).wait()

    def _scatter(j, b):
        return pltpu.async_copy(rows[b], acc.at[idx_d.at[j]], ssem[b],
                                add=True)

    def _scatter_wait(j, b):
        pltpu.make_async_copy(rows[b], acc.at[idx_d.at[j]], ssem[b]).wait()

    for k in range(_DEPTH - 1):
        _gather(k, k)
    _gather_wait(0, 0)
    _scatter(0, 0)
    _gather(_DEPTH - 1, _DEPTH - 1)

    def _step(t, carry):
        j0 = 1 + t * _DEPTH
        for u in range(_DEPTH):
            j = j0 + u
            b = (1 + u) % _DEPTH
            bn = u % _DEPTH
            _gather_wait(j, b)
            _scatter(j, b)
            _scatter_wait(j - 1, bn)
            _gather(j + _DEPTH - 1, bn)
        return carry

    nsteps = (CH - _DEPTH) // _DEPTH
    lax.fori_loop(0, nsteps, _step, 0)
    for j in range(CH - _DEPTH + 1, CH):
        _gather_wait(j, j % _DEPTH)
        _scatter(j, j % _DEPTH)
    for j in range(CH - _DEPTH, CH):
        _scatter_wait(j, j % _DEPTH)

    plsc.subcore_barrier()
    pltpu.sync_copy(acc.at[pl.ds(s * RPT, RPT)],
                    out_hbm.at[c, pl.ds(s * RPT, RPT)])


@functools.cache
def _make_segment_sum_sc(F):
    mesh = plsc.VectorSubcoreMesh(core_axis_name="c", subcore_axis_name="s")
    return functools.partial(
        pl.kernel,
        out_type=jax.ShapeDtypeStruct((NC, NPAD, F), jnp.float32),
        mesh=mesh,
        scratch_types=(
            [pltpu.VMEM((CH, CLEN), jnp.int32)] * 2
            + [pltpu.VMEM((CLEN, F), jnp.float32)] * _DEPTH
            + [pltpu.VMEM_SHARED((NPAD, F), jnp.float32),
               pltpu.VMEM_SHARED((N, F), jnp.float32)]
            + [pltpu.SemaphoreType.DMA] * (2 * _DEPTH + 1)
        ),
    )(_seg_body)


def _segment_sum_sc(q, src4d, dst4d, zeros, F):
    return _make_segment_sum_sc(F)(q, src4d, dst4d, zeros)


def _mm0_body(x_ref, ws_ref, wn_ref, b_ref, p_ref, q_ref):
    x = x_ref[...]
    p_ref[...] = (jnp.dot(x, ws_ref[...], preferred_element_type=jnp.float32)
                  + b_ref[...])
    q_ref[...] = jnp.dot(x, wn_ref[...], preferred_element_type=jnp.float32)


def _mid_body(p_ref, s_ref, ws_ref, wn_ref, b_ref, po_ref, qo_ref):
    h = jnp.maximum(p_ref[...] + s_ref[0], 0.0)
    po_ref[...] = (jnp.dot(h, ws_ref[...], preferred_element_type=jnp.float32)
                   + b_ref[...])
    qo_ref[...] = jnp.dot(h, wn_ref[...], preferred_element_type=jnp.float32)


def _fin_body(p_ref, s_ref, h_ref, g_ref):
    h = jnp.maximum(p_ref[...] + s_ref[0], 0.0)
    h_ref[...] = h
    col = jnp.sum(h, axis=0, keepdims=True)
    i = pl.program_id(0)

    @pl.when(i == 0)
    def _():
        g_ref[...] = jnp.zeros((2, OP), jnp.float32)

    row = jax.lax.broadcasted_iota(jnp.int32, (2, OP), 0)
    g_ref[...] = g_ref[...] + jnp.where(row == i // NBLK, col, 0.0)


def _attn_body(hs_ref, ht_ref, o_ref):
    sc = jax.lax.dot_general(hs_ref[...], ht_ref[...],
                             (((1,), (1,)), ((), ())),
                             preferred_element_type=jnp.float32)
    sc = sc * (1.0 / (O ** 0.5))
    m = jnp.max(sc, axis=-1, keepdims=True)
    e = jnp.exp(sc - m)
    o_ref[...] = e / jnp.sum(e, axis=-1, keepdims=True)


def _head_body(g_ref, wa_ref, wb_ref, b1_ref, w2_ref, b2_ref, sim_ref, lg_ref):
    g = g_ref[...] * (1.0 / N)
    gs = g[0:1, :]
    gt = g[1:2, :]
    z = jnp.maximum(
        jnp.dot(gs, wa_ref[...], preferred_element_type=jnp.float32)
        + jnp.dot(gt, wb_ref[...], preferred_element_type=jnp.float32)
        + b1_ref[...], 0.0)
    lg_ref[...] = (jnp.dot(z, w2_ref[...], preferred_element_type=jnp.float32)
                   + b2_ref[...])
    num = jnp.sum(gs * gt, axis=1, keepdims=True)
    ns = jnp.sqrt(jnp.sum(gs * gs, axis=1, keepdims=True))
    nt = jnp.sqrt(jnp.sum(gt * gt, axis=1, keepdims=True))
    sim_ref[...] = num / (ns * nt + 1e-8)


def _full(shape):
    return pl.BlockSpec(shape, lambda i: tuple(0 for _ in shape))


def _mm0(x2, Ws, Wn, b):
    return pl.pallas_call(
        _mm0_body,
        grid=(2 * NBLK,),
        in_specs=[pl.BlockSpec((BR, F_IN), lambda i: (i, 0)),
                  _full((F_IN, H)), _full((F_IN, H)), _full((1, H))],
        out_specs=[pl.BlockSpec((BR, H), lambda i: (i, 0)),
                   pl.BlockSpec((BR, H), lambda i: (i, 0))],
        out_shape=[jax.ShapeDtypeStruct((2 * N, H), jnp.float32)] * 2,
    )(x2, Ws, Wn, b.reshape(1, H))


def _mm_mid(p, seg, Ws, Wn, b, F_in, F_out):
    return pl.pallas_call(
        _mid_body,
        grid=(2 * NBLK,),
        in_specs=[pl.BlockSpec((BR, F_in), lambda i: (i, 0)),
                  pl.BlockSpec((1, BR, F_in),
                               lambda i: (i // NBLK, i % NBLK, 0)),
                  _full((F_in, F_out)), _full((F_in, F_out)),
                  _full((1, F_out))],
        out_specs=[pl.BlockSpec((BR, F_out), lambda i: (i, 0)),
                   pl.BlockSpec((BR, F_out), lambda i: (i, 0))],
        out_shape=[jax.ShapeDtypeStruct((2 * N, F_out), jnp.float32)] * 2,
    )(p, seg, Ws, Wn, b.reshape(1, F_out))


def _finalize(p, seg):
    return pl.pallas_call(
        _fin_body,
        grid=(2 * NBLK,),
        in_specs=[pl.BlockSpec((BR, OP), lambda i: (i, 0)),
                  pl.BlockSpec((1, BR, OP),
                               lambda i: (i // NBLK, i % NBLK, 0))],
        out_specs=[pl.BlockSpec((BR, OP), lambda i: (i, 0)),
                   pl.BlockSpec((2, OP), lambda i: (0, 0))],
        out_shape=[jax.ShapeDtypeStruct((2 * N, OP), jnp.float32),
                   jax.ShapeDtypeStruct((2, OP), jnp.float32)],
    )(p, seg)


BR_AT = 200


def _attention(h):
    return pl.pallas_call(
        _attn_body,
        grid=(N // BR_AT,),
        in_specs=[pl.BlockSpec((BR_AT, OP), lambda i: (i, 0)),
                  pl.BlockSpec((N, OP), lambda i: (1, 0))],
        out_specs=pl.BlockSpec((BR_AT, N), lambda i: (i, 0)),
        out_shape=jax.ShapeDtypeStruct((N, N), jnp.float32),
    )(h, h)


def _head(gsum, Wc1, bc1, Wc2, bc2):
    pad = ((0, OP - O), (0, 0))
    return pl.pallas_call(
        _head_body,
        out_shape=[jax.ShapeDtypeStruct((1, 1), jnp.float32),
                   jax.ShapeDtypeStruct((1, 4), jnp.float32)],
    )(gsum, jnp.pad(Wc1[:O], pad), jnp.pad(Wc1[O:], pad),
      bc1.reshape(1, H), Wc2, bc2.reshape(1, 4))


def kernel(x_s, edge_index_s, x_t, edge_index_t,
           W0s, W0n, b0, W1s, W1n, b1, W2s, W2n, b2,
           Wc1, bc1, Wc2, bc2):
    x2 = jnp.concatenate([x_s, x_t], axis=0)

    def _prep(edge_index):
        src = edge_index[0].astype(jnp.int32)
        dst = edge_index[1].astype(jnp.int32)
        src_p = jnp.concatenate([src, jnp.zeros((EPAD - E,), jnp.int32)])
        dst_p = jnp.concatenate([dst, jnp.full((EPAD - E,), N, jnp.int32)])
        return src_p.reshape(NS, CH, CLEN), dst_p.reshape(NS, CH, CLEN)

    ss, ds = _prep(edge_index_s)
    st, dt = _prep(edge_index_t)
    src4d = jnp.stack([ss, st])
    dst4d = jnp.stack([ds, dt])

    zeros = jnp.zeros((CLEN, H), jnp.float32)
    wpad = ((0, 0), (0, OP - O))
    p0, q0 = _mm0(x2, W0s, W0n, b0)
    seg0 = _segment_sum_sc(q0, src4d, dst4d, zeros, H)
    p1, q1 = _mm_mid(p0, seg0, W1s, W1n, b1, H, H)
    seg1 = _segment_sum_sc(q1, src4d, dst4d, zeros, H)
    p2, q2 = _mm_mid(p1, seg1, jnp.pad(W2s, wpad), jnp.pad(W2n, wpad),
                     jnp.pad(b2, (0, OP - O)), H, OP)
    seg2 = _segment_sum_sc(q2, src4d, dst4d, zeros, OP)
    h, gsum = _finalize(p2, seg2)

    attn = _attention(h)
    sim, logits = _head(gsum, Wc1, bc1, Wc2, bc2)
    return sim.reshape(()), attn, logits.reshape(4)
